# baseline (speedup 1.0000x reference)
import functools

import jax
import jax.numpy as jnp
from jax import lax
from jax.experimental import pallas as pl
from jax.experimental.pallas import tpu as pltpu
from jax.experimental.pallas import tpu_sc as plsc

NCLU = 16
NLEV = 128
NTOT = 4194304
NC, NS, L = 2, 16, 16
NW = NC * NS
CHUNK = NTOT // NW
SUB = 8192
NSUB = CHUNK // SUB
NACC = 8

TC_N = (NTOT * 3) // 4
SC_N = NTOT - TC_N
MM_CHUNK = SC_N // NW
MM_NSUB = MM_CHUNK // SUB
TC_ROWS = TC_N // 128
TC_STEP = 1024

_L2C = (-3.0283174810522704, 6.06583014324084, -5.264110477180775,
        3.218832837151793, -1.2342631730840539, 0.2668588228733046,
        -0.02482560661573763)

_MESH = plsc.VectorSubcoreMesh(core_axis_name="c", subcore_axis_name="s",
                               num_cores=NC, num_subcores=NS)
_SC_PARAMS = pltpu.CompilerParams(needs_layout_passes=False)

_F32 = jnp.float32
_I32 = jnp.int32


def _worker_id():
    return lax.axis_index("s") * NC + lax.axis_index("c")


def _stream_chunks(w_hbm, a_hbm, base, bufs, sems, process, carry_init,
                   nsub=NSUB):
    (wb0, ab0, wb1, ab1) = bufs
    (sw0, sa0, sw1, sa1) = sems

    pltpu.async_copy(w_hbm.at[pl.ds(base, SUB)], wb0, sw0)
    pltpu.async_copy(a_hbm.at[pl.ds(base, SUB)], ab0, sa0)

    def outer(io, carry):
        off0 = base + (2 * io) * SUB
        off1 = base + (2 * io + 1) * SUB
        off2 = base + (2 * io + 2) * SUB
        pltpu.async_copy(w_hbm.at[pl.ds(off1, SUB)], wb1, sw1)
        pltpu.async_copy(a_hbm.at[pl.ds(off1, SUB)], ab1, sa1)
        pltpu.make_async_copy(w_hbm.at[pl.ds(off0, SUB)], wb0, sw0).wait()
        pltpu.make_async_copy(a_hbm.at[pl.ds(off0, SUB)], ab0, sa0).wait()
        carry = process(wb0, ab0, carry)

        @pl.when(io < nsub // 2 - 1)
        def _():
            pltpu.async_copy(w_hbm.at[pl.ds(off2, SUB)], wb0, sw0)
            pltpu.async_copy(a_hbm.at[pl.ds(off2, SUB)], ab0, sa0)

        pltpu.make_async_copy(w_hbm.at[pl.ds(off1, SUB)], wb1, sw1).wait()
        pltpu.make_async_copy(a_hbm.at[pl.ds(off1, SUB)], ab1, sa1).wait()
        carry = process(wb1, ab1, carry)
        return carry

    return lax.fori_loop(0, nsub // 2, outer, carry_init)


def _minmax_body(w_hbm, a_hbm, omin_hbm, omax_hbm, *refs):
    wb0, ab0, wb1, ab1 = refs[:4]
    mns = refs[4:4 + NACC]
    mxs = refs[4 + NACC:4 + 2 * NACC]
    res = refs[4 + 2 * NACC]
    sw0, sa0, sw1, sa1 = refs[5 + 2 * NACC:9 + 2 * NACC]
    wid = _worker_id()
    base = wid * MM_CHUNK
    lane = lax.iota(_I32, L)

    def init(j, c):
        sl = pl.ds(j * L, L)
        for r in range(NACC):
            mns[r][sl] = jnp.full((L,), jnp.inf, _F32)
            mxs[r][sl] = jnp.full((L,), -jnp.inf, _F32)
        return c

    lax.fori_loop(0, NCLU, init, 0)

    def process(wb, ab, carry):
        def vec(k, c):
            for r in range(NACC):
                off = (k * NACC + r) * L
                w = wb[pl.ds(off, L)]
                a = ab[pl.ds(off, L)]
                idx = a * L + lane
                cur = plsc.load_gather(mns[r], [idx])
                plsc.store_scatter(mns[r], [idx], jnp.minimum(cur, w))
                cur = plsc.load_gather(mxs[r], [idx])
                plsc.store_scatter(mxs[r], [idx], jnp.maximum(cur, w))
            return c

        return lax.fori_loop(0, SUB // (NACC * L), vec, carry)

    _stream_chunks(w_hbm, a_hbm, base, (wb0, ab0, wb1, ab1),
                   (sw0, sa0, sw1, sa1), process, 0, nsub=MM_NSUB)

    for j in range(NCLU):
        sl = pl.ds(j * L, L)
        mn = mns[0][sl]
        mx = mxs[0][sl]
        for r in range(1, NACC):
            mn = jnp.minimum(mn, mns[r][sl])
            mx = jnp.maximum(mx, mxs[r][sl])
        mns[0][sl] = mn
        mxs[0][sl] = mx

    col = lane * L
    rmin = plsc.load_gather(mns[0], [col])
    rmax = plsc.load_gather(mxs[0], [col])
    for l in range(1, L):
        rmin = jnp.minimum(rmin, plsc.load_gather(mns[0], [col + l]))
        rmax = jnp.maximum(rmax, plsc.load_gather(mxs[0], [col + l]))
    res[pl.ds(0, L)] = rmin
    pltpu.sync_copy(res, omin_hbm.at[pl.ds(wid * L, L)])
    res[pl.ds(0, L)] = rmax
    pltpu.sync_copy(res, omax_hbm.at[pl.ds(wid * L, L)])


_minmax_call = pl.kernel(
    _minmax_body,
    out_type=[jax.ShapeDtypeStruct((NW * L,), _F32),
              jax.ShapeDtypeStruct((NW * L,), _F32)],
    mesh=_MESH,
    compiler_params=_SC_PARAMS,
    scratch_types=[
        pltpu.VMEM((SUB,), _F32), pltpu.VMEM((SUB,), _I32),
        pltpu.VMEM((SUB,), _F32), pltpu.VMEM((SUB,), _I32),
    ] + [pltpu.VMEM((NCLU * L,), _F32)] * (2 * NACC) + [
        pltpu.VMEM((L,), _F32),
        pltpu.SemaphoreType.DMA, pltpu.SemaphoreType.DMA,
        pltpu.SemaphoreType.DMA, pltpu.SemaphoreType.DMA,
    ],
)


def _tcmm_body(w_ref, a_ref, omin_ref, omax_ref):
    @pl.when(pl.program_id(0) == 0)
    def _():
        omin_ref[:, :] = jnp.full((NCLU, 128), jnp.inf, _F32)
        omax_ref[:, :] = jnp.full((NCLU, 128), -jnp.inf, _F32)

    w = w_ref[:, :]
    a = a_ref[:, :]
    for c in range(NCLU):
        mask = a == c
        mn = jnp.min(jnp.where(mask, w, jnp.inf), axis=0, keepdims=True)
        mx = jnp.max(jnp.where(mask, w, -jnp.inf), axis=0, keepdims=True)
        omin_ref[pl.ds(c, 1), :] = jnp.minimum(omin_ref[pl.ds(c, 1), :], mn)
        omax_ref[pl.ds(c, 1), :] = jnp.maximum(omax_ref[pl.ds(c, 1), :], mx)


_tcmm_call = pl.pallas_call(
    _tcmm_body,
    grid=(TC_ROWS // TC_STEP,),
    in_specs=[pl.BlockSpec((TC_STEP, 128), lambda i: (i, 0)),
              pl.BlockSpec((TC_STEP, 128), lambda i: (i, 0))],
    out_specs=[pl.BlockSpec((NCLU, 128), lambda i: (0, 0)),
               pl.BlockSpec((NCLU, 128), lambda i: (0, 0))],
    out_shape=[jax.ShapeDtypeStruct((NCLU, 128), _F32),
               jax.ShapeDtypeStruct((NCLU, 128), _F32)],
)


def _diag_col(row, fill):
    b = jnp.broadcast_to(row, (NCLU, NCLU))
    ii = lax.broadcasted_iota(_I32, (NCLU, NCLU), 0)
    jj = lax.broadcasted_iota(_I32, (NCLU, NCLU), 1)
    return jnp.min(jnp.where(ii == jj, b, fill), axis=1, keepdims=True)


def _diag_row(colv, fill):
    b = jnp.broadcast_to(colv, (NCLU, NCLU))
    ii = lax.broadcasted_iota(_I32, (NCLU, NCLU), 0)
    jj = lax.broadcasted_iota(_I32, (NCLU, NCLU), 1)
    return jnp.min(jnp.where(ii == jj, b, fill), axis=0, keepdims=True)


def _table_body(pmin_ref, pmax_ref, tcmin_ref, tcmax_ref,
                bits_row_ref, bits_col_ref, logits_ref,
                table_ref, s_ref, t_ref, nlm1_ref):
    tmin = _diag_row(jnp.min(tcmin_ref[:], axis=1, keepdims=True), jnp.inf)
    tmax = -_diag_row(-jnp.max(tcmax_ref[:], axis=1, keepdims=True), jnp.inf)
    gmin = jnp.minimum(jnp.min(pmin_ref[:], axis=0, keepdims=True), tmin)
    gmax = jnp.maximum(jnp.max(pmax_ref[:], axis=0, keepdims=True), tmax)
    rng = gmax - gmin
    gate = rng >= 1e-8
    nl = jnp.left_shift(1, bits_row_ref[:])
    nlm1f = (nl - 1).astype(_F32)
    s = jnp.where(gate, nlm1f / jnp.where(gate, rng, 1.0), 0.0)
    s_ref[:, :] = s
    t_ref[:, :] = -jnp.where(gate, gmin, 0.0) * s
    nlm1_ref[:, :] = jnp.where(gate, nlm1f, 0.0)

    gmin_c = _diag_col(gmin, jnp.inf)
    gmax_c = -_diag_col(-gmax, jnp.inf)
    gate_c = (gmax_c - gmin_c) >= 1e-8
    nl_c = jnp.left_shift(1, bits_col_ref[:])
    logits = logits_ref[:, :NLEV]
    colid = lax.broadcasted_iota(_I32, (NCLU, NLEV), 1)
    lmask = colid < nl_c
    lm = jnp.where(lmask, logits, -jnp.inf)
    mx = jnp.max(lm, axis=1, keepdims=True)
    e = jnp.exp(lm - mx)
    p = e / jnp.sum(e, axis=1, keepdims=True)
    last = jnp.sum(jnp.where(colid == nl_c - 1, p, 0.0), axis=1,
                   keepdims=True)
    pfull = jnp.where(lmask, p, jnp.broadcast_to(last, (NCLU, NLEV)))
    pfull = jnp.where(gate_c, pfull, 1.0) + 1e-10
    ii = lax.broadcasted_iota(_I32, (NCLU, NCLU), 0)
    jj = lax.broadcasted_iota(_I32, (NCLU, NCLU), 1)
    eye = (ii == jj).astype(_F32)
    dn = (((0,), (0,)), ((), ()))
    table_ref[pl.ds(0, NLEV), :] = lax.dot_general(pfull, eye, dn)
    lastp = jnp.where(gate_c, last, 1.0) + 1e-10
    table_ref[pl.ds(NLEV, 1), :] = lax.dot_general(lastp, eye, dn)


def _table_call(pmin, pmax, tcmin, tcmax, bits, logits):
    table, s, t, nlm1 = pl.pallas_call(
        _table_body,
        out_shape=[jax.ShapeDtypeStruct((NLEV + 1, NCLU), _F32),
                   jax.ShapeDtypeStruct((1, NCLU), _F32),
                   jax.ShapeDtypeStruct((1, NCLU), _F32),
                   jax.ShapeDtypeStruct((1, NCLU), _F32)],
    )(pmin, pmax, tcmin, tcmax,
      bits.reshape(1, NCLU), bits.reshape(NCLU, 1), logits)
    return table, s.reshape(NCLU), t.reshape(NCLU), nlm1.reshape(NCLU)


def _log2(x):
    bi = plsc.bitcast(x, _I32)
    ex = (bi >> 23) - 127
    mant = plsc.bitcast((bi & 0x007FFFFF) | 0x3F800000, _F32)
    p = jnp.full((L,), _L2C[6], _F32)
    for c in (_L2C[5], _L2C[4], _L2C[3], _L2C[2], _L2C[1], _L2C[0]):
        p = p * mant + c
    return ex.astype(_F32) + p


def _main_body(w_hbm, a_hbm, tab_hbm, s_hbm, t_hbm, nlm1_hbm, out_hbm,
               wb0, ab0, wb1, ab1, tabv, sv, tv, nlv, res,
               sw0, sa0, sw1, sa1):
    wid = _worker_id()
    base = wid * CHUNK
    pltpu.sync_copy(tab_hbm, tabv)
    pltpu.sync_copy(s_hbm, sv)
    pltpu.sync_copy(t_hbm, tv)
    pltpu.sync_copy(nlm1_hbm, nlv)

    def one(wb, ab, off, acc):
        w = wb[pl.ds(off, L)]
        a = ab[pl.ds(off, L)]
        s = plsc.load_gather(sv, [a])
        t = plsc.load_gather(tv, [a])
        nf = plsc.load_gather(nlv, [a])
        soft = jnp.minimum(w * s + t, nf)
        ifl = soft.astype(_I32)
        alpha = soft - ifl.astype(_F32)
        addr = (ifl << 4) + a
        pf = plsc.load_gather(tabv, [addr])
        pc = plsc.load_gather(tabv, [addr + L])
        itp = pf + alpha * (pc - pf)
        return acc + _log2(itp)

    def process(wb, ab, accs):
        def vec(k, accs):
            a0, a1 = accs
            off = k * (2 * L)
            a0 = one(wb, ab, off, a0)
            a1 = one(wb, ab, off + L, a1)
            return (a0, a1)

        return lax.fori_loop(0, SUB // (2 * L), vec, accs)

    acc0, acc1 = _stream_chunks(
        w_hbm, a_hbm, base, (wb0, ab0, wb1, ab1),
        (sw0, sa0, sw1, sa1), process,
        (jnp.zeros((L,), _F32), jnp.zeros((L,), _F32)))
    res[pl.ds(0, L)] = acc0 + acc1
    pltpu.sync_copy(res, out_hbm.at[pl.ds(wid * L, L)])


_main_call = pl.kernel(
    _main_body,
    out_type=[jax.ShapeDtypeStruct((NW * L,), _F32)],
    mesh=_MESH,
    compiler_params=_SC_PARAMS,
    scratch_types=[
        pltpu.VMEM((SUB,), _F32), pltpu.VMEM((SUB,), _I32),
        pltpu.VMEM((SUB,), _F32), pltpu.VMEM((SUB,), _I32),
        pltpu.VMEM(((NLEV + 1) * NCLU,), _F32),
        pltpu.VMEM((NCLU,), _F32), pltpu.VMEM((NCLU,), _F32),
        pltpu.VMEM((NCLU,), _F32),
        pltpu.VMEM((L,), _F32),
        pltpu.SemaphoreType.DMA, pltpu.SemaphoreType.DMA,
        pltpu.SemaphoreType.DMA, pltpu.SemaphoreType.DMA,
    ],
)


def _final_body(part_ref, out_ref):
    t = jnp.sum(part_ref[:], axis=1, keepdims=True)
    out_ref[:, :] = -jnp.sum(t, axis=0, keepdims=True) / float(NTOT)


def _final_call(part):
    return pl.pallas_call(
        _final_body,
        out_shape=jax.ShapeDtypeStruct((1, 1), _F32),
    )(part.reshape(NW, L))


@jax.jit
def kernel(quantized_weights, cluster_assignments, bit_allocation,
           cluster_logits, cluster_scales):
    w = quantized_weights.reshape(NTOT)
    a = cluster_assignments.reshape(NTOT)
    omin, omax = _minmax_call(w, a)
    tcmin, tcmax = _tcmm_call(w[SC_N:].reshape(TC_ROWS, 128),
                              a[SC_N:].reshape(TC_ROWS, 128))
    table, s, t, nlm1 = _table_call(
        omin.reshape(NW, L), omax.reshape(NW, L), tcmin, tcmax,
        bit_allocation, cluster_logits)
    (part,) = _main_call(w, a, table.reshape((NLEV + 1) * NCLU), s, t, nlm1)
    return _final_call(part)[0, 0]


# speedup vs baseline: 1.0452x; 1.0452x over previous
import functools

import jax
import jax.numpy as jnp
from jax import lax
from jax.experimental import pallas as pl
from jax.experimental.pallas import tpu as pltpu
from jax.experimental.pallas import tpu_sc as plsc

NCLU = 16
NLEV = 128
NTOT = 4194304
NC, NS, L = 2, 16, 16
NW = NC * NS
CHUNK = NTOT // NW
SUB = 8192
NSUB = CHUNK // SUB
NACC = 8

TC_N = (NTOT * 5) // 8
SC_N = NTOT - TC_N
MM_CHUNK = SC_N // NW
MM_NSUB = MM_CHUNK // SUB
TC_ROWS = TC_N // 128
TC_STEP = 1024

_L2C = (-3.0283174810522704, 6.06583014324084, -5.264110477180775,
        3.218832837151793, -1.2342631730840539, 0.2668588228733046,
        -0.02482560661573763)

_MESH = plsc.VectorSubcoreMesh(core_axis_name="c", subcore_axis_name="s",
                               num_cores=NC, num_subcores=NS)
_SC_PARAMS = pltpu.CompilerParams(needs_layout_passes=False)

_F32 = jnp.float32
_I32 = jnp.int32


def _worker_id():
    return lax.axis_index("s") * NC + lax.axis_index("c")


def _stream_chunks(w_hbm, a_hbm, base, bufs, sems, process, carry_init,
                   nsub=NSUB):
    (wb0, ab0, wb1, ab1) = bufs
    (sw0, sa0, sw1, sa1) = sems

    pltpu.async_copy(w_hbm.at[pl.ds(base, SUB)], wb0, sw0)
    pltpu.async_copy(a_hbm.at[pl.ds(base, SUB)], ab0, sa0)

    def outer(io, carry):
        off0 = base + (2 * io) * SUB
        off1 = base + (2 * io + 1) * SUB
        off2 = base + (2 * io + 2) * SUB
        pltpu.async_copy(w_hbm.at[pl.ds(off1, SUB)], wb1, sw1)
        pltpu.async_copy(a_hbm.at[pl.ds(off1, SUB)], ab1, sa1)
        pltpu.make_async_copy(w_hbm.at[pl.ds(off0, SUB)], wb0, sw0).wait()
        pltpu.make_async_copy(a_hbm.at[pl.ds(off0, SUB)], ab0, sa0).wait()
        carry = process(wb0, ab0, carry)

        @pl.when(io < nsub // 2 - 1)
        def _():
            pltpu.async_copy(w_hbm.at[pl.ds(off2, SUB)], wb0, sw0)
            pltpu.async_copy(a_hbm.at[pl.ds(off2, SUB)], ab0, sa0)

        pltpu.make_async_copy(w_hbm.at[pl.ds(off1, SUB)], wb1, sw1).wait()
        pltpu.make_async_copy(a_hbm.at[pl.ds(off1, SUB)], ab1, sa1).wait()
        carry = process(wb1, ab1, carry)
        return carry

    return lax.fori_loop(0, nsub // 2, outer, carry_init)


def _minmax_body(w_hbm, a_hbm, omin_hbm, omax_hbm, *refs):
    wb0, ab0, wb1, ab1 = refs[:4]
    mns = refs[4:4 + NACC]
    mxs = refs[4 + NACC:4 + 2 * NACC]
    res = refs[4 + 2 * NACC]
    sw0, sa0, sw1, sa1 = refs[5 + 2 * NACC:9 + 2 * NACC]
    wid = _worker_id()
    base = wid * MM_CHUNK
    lane = lax.iota(_I32, L)

    def init(j, c):
        sl = pl.ds(j * L, L)
        for r in range(NACC):
            mns[r][sl] = jnp.full((L,), jnp.inf, _F32)
            mxs[r][sl] = jnp.full((L,), -jnp.inf, _F32)
        return c

    lax.fori_loop(0, NCLU, init, 0)

    def process(wb, ab, carry):
        def vec(k, c):
            for r in range(NACC):
                off = (k * NACC + r) * L
                w = wb[pl.ds(off, L)]
                a = ab[pl.ds(off, L)]
                idx = a * L + lane
                cur = plsc.load_gather(mns[r], [idx])
                plsc.store_scatter(mns[r], [idx], jnp.minimum(cur, w))
                cur = plsc.load_gather(mxs[r], [idx])
                plsc.store_scatter(mxs[r], [idx], jnp.maximum(cur, w))
            return c

        return lax.fori_loop(0, SUB // (NACC * L), vec, carry)

    _stream_chunks(w_hbm, a_hbm, base, (wb0, ab0, wb1, ab1),
                   (sw0, sa0, sw1, sa1), process, 0, nsub=MM_NSUB)

    for j in range(NCLU):
        sl = pl.ds(j * L, L)
        mn = mns[0][sl]
        mx = mxs[0][sl]
        for r in range(1, NACC):
            mn = jnp.minimum(mn, mns[r][sl])
            mx = jnp.maximum(mx, mxs[r][sl])
        mns[0][sl] = mn
        mxs[0][sl] = mx

    col = lane * L
    rmin = plsc.load_gather(mns[0], [col])
    rmax = plsc.load_gather(mxs[0], [col])
    for l in range(1, L):
        rmin = jnp.minimum(rmin, plsc.load_gather(mns[0], [col + l]))
        rmax = jnp.maximum(rmax, plsc.load_gather(mxs[0], [col + l]))
    res[pl.ds(0, L)] = rmin
    pltpu.sync_copy(res, omin_hbm.at[pl.ds(wid * L, L)])
    res[pl.ds(0, L)] = rmax
    pltpu.sync_copy(res, omax_hbm.at[pl.ds(wid * L, L)])


_minmax_call = pl.kernel(
    _minmax_body,
    out_type=[jax.ShapeDtypeStruct((NW * L,), _F32),
              jax.ShapeDtypeStruct((NW * L,), _F32)],
    mesh=_MESH,
    compiler_params=_SC_PARAMS,
    scratch_types=[
        pltpu.VMEM((SUB,), _F32), pltpu.VMEM((SUB,), _I32),
        pltpu.VMEM((SUB,), _F32), pltpu.VMEM((SUB,), _I32),
    ] + [pltpu.VMEM((NCLU * L,), _F32)] * (2 * NACC) + [
        pltpu.VMEM((L,), _F32),
        pltpu.SemaphoreType.DMA, pltpu.SemaphoreType.DMA,
        pltpu.SemaphoreType.DMA, pltpu.SemaphoreType.DMA,
    ],
)


def _tcmm_body(w_ref, a_ref, omin_ref, omax_ref):
    @pl.when(pl.program_id(0) == 0)
    def _():
        omin_ref[:, :] = jnp.full((NCLU, 128), jnp.inf, _F32)
        omax_ref[:, :] = jnp.full((NCLU, 128), -jnp.inf, _F32)

    w = w_ref[:, :]
    a = a_ref[:, :]
    for c in range(NCLU):
        mask = a == c
        mn = jnp.min(jnp.where(mask, w, jnp.inf), axis=0, keepdims=True)
        mx = jnp.max(jnp.where(mask, w, -jnp.inf), axis=0, keepdims=True)
        omin_ref[pl.ds(c, 1), :] = jnp.minimum(omin_ref[pl.ds(c, 1), :], mn)
        omax_ref[pl.ds(c, 1), :] = jnp.maximum(omax_ref[pl.ds(c, 1), :], mx)


_tcmm_call = pl.pallas_call(
    _tcmm_body,
    grid=(TC_ROWS // TC_STEP,),
    in_specs=[pl.BlockSpec((TC_STEP, 128), lambda i: (i, 0)),
              pl.BlockSpec((TC_STEP, 128), lambda i: (i, 0))],
    out_specs=[pl.BlockSpec((NCLU, 128), lambda i: (0, 0)),
               pl.BlockSpec((NCLU, 128), lambda i: (0, 0))],
    out_shape=[jax.ShapeDtypeStruct((NCLU, 128), _F32),
               jax.ShapeDtypeStruct((NCLU, 128), _F32)],
)


def _diag_col(row, fill):
    b = jnp.broadcast_to(row, (NCLU, NCLU))
    ii = lax.broadcasted_iota(_I32, (NCLU, NCLU), 0)
    jj = lax.broadcasted_iota(_I32, (NCLU, NCLU), 1)
    return jnp.min(jnp.where(ii == jj, b, fill), axis=1, keepdims=True)


def _diag_row(colv, fill):
    b = jnp.broadcast_to(colv, (NCLU, NCLU))
    ii = lax.broadcasted_iota(_I32, (NCLU, NCLU), 0)
    jj = lax.broadcasted_iota(_I32, (NCLU, NCLU), 1)
    return jnp.min(jnp.where(ii == jj, b, fill), axis=0, keepdims=True)


def _table_body(pmin_ref, pmax_ref, tcmin_ref, tcmax_ref,
                bits_row_ref, bits_col_ref, logits_ref,
                table_ref, s_ref, t_ref, nlm1_ref):
    tmin = _diag_row(jnp.min(tcmin_ref[:], axis=1, keepdims=True), jnp.inf)
    tmax = -_diag_row(-jnp.max(tcmax_ref[:], axis=1, keepdims=True), jnp.inf)
    gmin = jnp.minimum(jnp.min(pmin_ref[:], axis=0, keepdims=True), tmin)
    gmax = jnp.maximum(jnp.max(pmax_ref[:], axis=0, keepdims=True), tmax)
    rng = gmax - gmin
    gate = rng >= 1e-8
    nl = jnp.left_shift(1, bits_row_ref[:])
    nlm1f = (nl - 1).astype(_F32)
    s = jnp.where(gate, nlm1f / jnp.where(gate, rng, 1.0), 0.0)
    s_ref[:, :] = s
    t_ref[:, :] = -jnp.where(gate, gmin, 0.0) * s
    nlm1_ref[:, :] = jnp.where(gate, nlm1f, 0.0)

    gmin_c = _diag_col(gmin, jnp.inf)
    gmax_c = -_diag_col(-gmax, jnp.inf)
    gate_c = (gmax_c - gmin_c) >= 1e-8
    nl_c = jnp.left_shift(1, bits_col_ref[:])
    logits = logits_ref[:, :NLEV]
    colid = lax.broadcasted_iota(_I32, (NCLU, NLEV), 1)
    lmask = colid < nl_c
    lm = jnp.where(lmask, logits, -jnp.inf)
    mx = jnp.max(lm, axis=1, keepdims=True)
    e = jnp.exp(lm - mx)
    p = e / jnp.sum(e, axis=1, keepdims=True)
    last = jnp.sum(jnp.where(colid == nl_c - 1, p, 0.0), axis=1,
                   keepdims=True)
    pfull = jnp.where(lmask, p, jnp.broadcast_to(last, (NCLU, NLEV)))
    pfull = jnp.where(gate_c, pfull, 1.0) + 1e-10
    ii = lax.broadcasted_iota(_I32, (NCLU, NCLU), 0)
    jj = lax.broadcasted_iota(_I32, (NCLU, NCLU), 1)
    eye = (ii == jj).astype(_F32)
    dn = (((0,), (0,)), ((), ()))
    table_ref[pl.ds(0, NLEV), :] = lax.dot_general(pfull, eye, dn)
    lastp = jnp.where(gate_c, last, 1.0) + 1e-10
    table_ref[pl.ds(NLEV, 1), :] = lax.dot_general(lastp, eye, dn)


def _table_call(pmin, pmax, tcmin, tcmax, bits, logits):
    table, s, t, nlm1 = pl.pallas_call(
        _table_body,
        out_shape=[jax.ShapeDtypeStruct((NLEV + 1, NCLU), _F32),
                   jax.ShapeDtypeStruct((1, NCLU), _F32),
                   jax.ShapeDtypeStruct((1, NCLU), _F32),
                   jax.ShapeDtypeStruct((1, NCLU), _F32)],
    )(pmin, pmax, tcmin, tcmax,
      bits.reshape(1, NCLU), bits.reshape(NCLU, 1), logits)
    return table, s.reshape(NCLU), t.reshape(NCLU), nlm1.reshape(NCLU)


def _log2(x):
    bi = plsc.bitcast(x, _I32)
    ex = (bi >> 23) - 127
    mant = plsc.bitcast((bi & 0x007FFFFF) | 0x3F800000, _F32)
    p = jnp.full((L,), _L2C[6], _F32)
    for c in (_L2C[5], _L2C[4], _L2C[3], _L2C[2], _L2C[1], _L2C[0]):
        p = p * mant + c
    return ex.astype(_F32) + p


def _main_body(w_hbm, a_hbm, tab_hbm, s_hbm, t_hbm, nlm1_hbm, out_hbm,
               wb0, ab0, wb1, ab1, tabv, sv, tv, nlv, res,
               sw0, sa0, sw1, sa1):
    wid = _worker_id()
    base = wid * CHUNK
    pltpu.sync_copy(tab_hbm, tabv)
    pltpu.sync_copy(s_hbm, sv)
    pltpu.sync_copy(t_hbm, tv)
    pltpu.sync_copy(nlm1_hbm, nlv)

    def one(wb, ab, off, acc):
        w = wb[pl.ds(off, L)]
        a = ab[pl.ds(off, L)]
        s = plsc.load_gather(sv, [a])
        t = plsc.load_gather(tv, [a])
        nf = plsc.load_gather(nlv, [a])
        soft = jnp.minimum(w * s + t, nf)
        ifl = soft.astype(_I32)
        alpha = soft - ifl.astype(_F32)
        addr = (ifl << 4) + a
        pf = plsc.load_gather(tabv, [addr])
        pc = plsc.load_gather(tabv, [addr + L])
        itp = pf + alpha * (pc - pf)
        return acc + _log2(itp)

    def process(wb, ab, accs):
        def vec(k, accs):
            a0, a1 = accs
            off = k * (2 * L)
            a0 = one(wb, ab, off, a0)
            a1 = one(wb, ab, off + L, a1)
            return (a0, a1)

        return lax.fori_loop(0, SUB // (2 * L), vec, accs)

    acc0, acc1 = _stream_chunks(
        w_hbm, a_hbm, base, (wb0, ab0, wb1, ab1),
        (sw0, sa0, sw1, sa1), process,
        (jnp.zeros((L,), _F32), jnp.zeros((L,), _F32)))
    res[pl.ds(0, L)] = acc0 + acc1
    pltpu.sync_copy(res, out_hbm.at[pl.ds(wid * L, L)])


_main_call = pl.kernel(
    _main_body,
    out_type=[jax.ShapeDtypeStruct((NW * L,), _F32)],
    mesh=_MESH,
    compiler_params=_SC_PARAMS,
    scratch_types=[
        pltpu.VMEM((SUB,), _F32), pltpu.VMEM((SUB,), _I32),
        pltpu.VMEM((SUB,), _F32), pltpu.VMEM((SUB,), _I32),
        pltpu.VMEM(((NLEV + 1) * NCLU,), _F32),
        pltpu.VMEM((NCLU,), _F32), pltpu.VMEM((NCLU,), _F32),
        pltpu.VMEM((NCLU,), _F32),
        pltpu.VMEM((L,), _F32),
        pltpu.SemaphoreType.DMA, pltpu.SemaphoreType.DMA,
        pltpu.SemaphoreType.DMA, pltpu.SemaphoreType.DMA,
    ],
)


def _final_body(part_ref, out_ref):
    t = jnp.sum(part_ref[:], axis=1, keepdims=True)
    out_ref[:, :] = -jnp.sum(t, axis=0, keepdims=True) / float(NTOT)


def _final_call(part):
    return pl.pallas_call(
        _final_body,
        out_shape=jax.ShapeDtypeStruct((1, 1), _F32),
    )(part.reshape(NW, L))


@jax.jit
def kernel(quantized_weights, cluster_assignments, bit_allocation,
           cluster_logits, cluster_scales):
    w = quantized_weights.reshape(NTOT)
    a = cluster_assignments.reshape(NTOT)
    omin, omax = _minmax_call(w, a)
    tcmin, tcmax = _tcmm_call(w[SC_N:].reshape(TC_ROWS, 128),
                              a[SC_N:].reshape(TC_ROWS, 128))
    table, s, t, nlm1 = _table_call(
        omin.reshape(NW, L), omax.reshape(NW, L), tcmin, tcmax,
        bit_allocation, cluster_logits)
    (part,) = _main_call(w, a, table.reshape((NLEV + 1) * NCLU), s, t, nlm1)
    return _final_call(part)[0, 0]
